# Initial kernel scaffold; baseline (speedup 1.0000x reference)
import jax
import jax.numpy as jnp
from jax.experimental import pallas as pl


def kernel(decoder_feat, codebook, positive_indices):
    raise NotImplementedError("write your pallas kernel here")


# speedup vs baseline: 17.5400x; 17.5400x over previous
import jax
import jax.numpy as jnp
from jax import lax
from jax.experimental import pallas as pl
from jax.experimental.pallas import tpu as pltpu
from jax.experimental.pallas import tpu_sc as plsc

_TEMP = 0.07
_NUM_HARD = 16
_TOPK = _NUM_HARD + 1

_SC_CORES = 2
_SC_SUBCORES = 16
_SC_WORKERS = _SC_CORES * _SC_SUBCORES
_GATHER_CHUNK = 128


def _gather_rows_sc(table, idx):
    K, C = table.shape
    N = idx.shape[0]
    b_per_w = N // _SC_WORKERS
    chunk = min(_GATHER_CHUNK, b_per_w)
    n_chunks = b_per_w // chunk
    mesh = plsc.VectorSubcoreMesh(core_axis_name="c", subcore_axis_name="s")

    def body(table_hbm, idx_hbm, out_hbm, idx_v, rows_v, sem):
        wid = lax.axis_index("s") * _SC_CORES + lax.axis_index("c")
        base = wid * b_per_w
        pltpu.sync_copy(idx_hbm.at[pl.ds(base, b_per_w)], idx_v)
        copies = []
        for j in range(n_chunks):
            copies.append(pltpu.async_copy(
                table_hbm.at[idx_v.at[pl.ds(j * chunk, chunk)]],
                rows_v.at[pl.ds(j * chunk, chunk), :],
                sem,
            ))
        for cp in copies:
            cp.wait()
        pltpu.sync_copy(rows_v, out_hbm.at[pl.ds(base, b_per_w)])

    return pl.kernel(
        body,
        out_type=jax.ShapeDtypeStruct((N, C), table.dtype),
        mesh=mesh,
        compiler_params=pltpu.CompilerParams(use_tc_tiling_on_sc=False),
        scratch_types=[
            pltpu.VMEM((b_per_w,), jnp.int32),
            pltpu.VMEM((b_per_w, C), jnp.float32),
            pltpu.SemaphoreType.DMA,
        ],
    )(table, idx)


def _topk_body(pos_ref, feat_ref, cbt_ref, hard_ref, pos_logit_ref, q_ref):
    K = cbt_ref.shape[1]
    pos = pos_ref[...]
    cbt = cbt_ref[...]
    a2 = jnp.sum(pos * pos, axis=1, keepdims=True)
    b2 = jnp.sum(cbt * cbt, axis=0, keepdims=True)
    dot = jnp.dot(pos, cbt, preferred_element_type=jnp.float32,
                  precision=lax.Precision.HIGHEST)
    s = jnp.maximum(a2 + b2 - 2.0 * dot, 0.0)
    iota = lax.broadcasted_iota(jnp.int32, s.shape, 1)
    keys = (lax.bitcast_convert_type(s, jnp.int32) & jnp.int32(~(K - 1))) | iota
    m = jnp.min(keys, axis=1, keepdims=True)
    c2 = m + jnp.int32(1) + jnp.int32(-2147483648)
    cols = []
    for _ in range(_NUM_HARD):
        mz = jnp.min(keys - c2, axis=1, keepdims=True)
        nxt = mz + c2
        cols.append(nxt & jnp.int32(K - 1))
        c2 = nxt + jnp.int32(1) + jnp.int32(-2147483648)
    hard_ref[...] = jnp.concatenate(cols, axis=1)
    feat = feat_ref[...]
    qn = jnp.sqrt(jnp.sum(feat * feat, axis=1, keepdims=True))
    q = feat / jnp.maximum(qn, 1e-12)
    kp = pos / jnp.maximum(jnp.sqrt(a2), 1e-12)
    q_ref[...] = q
    pos_logit_ref[...] = jnp.sum(q * kp, axis=1, keepdims=True) / _TEMP


def _loss_body(q_ref, pos_l_ref, neg_ref, out_ref):
    q = q_ref[...]
    neg = neg_ref[...]
    C = q.shape[1]
    M = neg.shape[1]
    cc = lax.broadcasted_iota(jnp.int32, (C, M), 0)
    mm = lax.broadcasted_iota(jnp.int32, (C, M), 1)
    J = (mm % C == cc).astype(jnp.float32)
    m1 = lax.broadcasted_iota(jnp.int32, (M, _NUM_HARD), 0)
    n1 = lax.broadcasted_iota(jnp.int32, (M, _NUM_HARD), 1)
    S = (m1 // C == n1).astype(jnp.float32)
    qt = jnp.dot(q, J, preferred_element_type=jnp.float32,
                 precision=lax.Precision.HIGHEST)
    dots = jnp.dot(neg * qt, S, preferred_element_type=jnp.float32,
                   precision=lax.Precision.HIGHEST)
    n2 = jnp.dot(neg * neg, S, preferred_element_type=jnp.float32,
                 precision=lax.Precision.HIGHEST)
    nl = dots / jnp.maximum(jnp.sqrt(n2), 1e-12) / _TEMP
    pos_l = pos_l_ref[...]
    logits = jnp.concatenate([pos_l, nl], axis=1)
    m = jnp.max(logits, axis=1, keepdims=True)
    lse = jnp.log(jnp.sum(jnp.exp(logits - m), axis=1, keepdims=True)) + m
    out_ref[...] = jnp.mean(lse - pos_l).reshape(1, 1)


def kernel(decoder_feat, codebook, positive_indices):
    B, L, C = decoder_feat.shape
    K = codebook.shape[0]
    BL = B * L
    feat = decoder_feat.reshape(BL, C)
    pos_flat = jnp.clip(positive_indices, 0, K - 1).reshape(BL).astype(jnp.int32)

    pos_emb = _gather_rows_sc(codebook, pos_flat)

    R = 512
    hard, pos_logit, q = pl.pallas_call(
        _topk_body,
        grid=(BL // R,),
        in_specs=[
            pl.BlockSpec((R, C), lambda i: (i, 0)),
            pl.BlockSpec((R, C), lambda i: (i, 0)),
            pl.BlockSpec((C, K), lambda i: (0, 0)),
        ],
        out_specs=[
            pl.BlockSpec((R, _NUM_HARD), lambda i: (i, 0)),
            pl.BlockSpec((R, 1), lambda i: (i, 0)),
            pl.BlockSpec((R, C), lambda i: (i, 0)),
        ],
        out_shape=[
            jax.ShapeDtypeStruct((BL, _NUM_HARD), jnp.int32),
            jax.ShapeDtypeStruct((BL, 1), jnp.float32),
            jax.ShapeDtypeStruct((BL, C), jnp.float32),
        ],
    )(pos_emb, feat, codebook.T)

    neg = _gather_rows_sc(codebook, hard.reshape(BL * _NUM_HARD))

    loss = pl.pallas_call(
        _loss_body,
        in_specs=[
            pl.BlockSpec((BL, C), lambda: (0, 0)),
            pl.BlockSpec((BL, 1), lambda: (0, 0)),
            pl.BlockSpec((BL, _NUM_HARD * C), lambda: (0, 0)),
        ],
        out_specs=pl.BlockSpec((1, 1), lambda: (0, 0)),
        out_shape=jax.ShapeDtypeStruct((1, 1), jnp.float32),
    )(q, pos_logit, neg.reshape(BL, _NUM_HARD * C))

    return loss[0, 0]
